# baseline (speedup 1.0000x reference)
import jax
import jax.numpy as jnp
from jax import lax
from jax.experimental import pallas as pl
from jax.experimental.pallas import tpu as pltpu
from jax.experimental.pallas import tpu_sc as plsc

D_MODEL = 1024
SEQ_LEN = 4096
BATCH = 4
NUM_CORES = 2
NUM_SUBCORES = 16
NUM_WORKERS = NUM_CORES * NUM_SUBCORES
SEQ_PER_WORKER = SEQ_LEN // NUM_WORKERS
CHUNK_ROWS = 16
NUM_CHUNKS = SEQ_PER_WORKER // CHUNK_ROWS
LANES = 16
UNROLL = 4

TC_BS = 2048


def _sc_body(x_hbm, emb_hbm, out_hbm,
             xb0, xb1, xb2, xb3, eb0, eb1, eb2, eb3,
             ls0, ls1, ls2, ls3, es0, es1, es2, es3,
             ss0, ss1, ss2, ss3):
    xbufs = [xb0, xb1, xb2, xb3]
    ebufs = [eb0, eb1, eb2, eb3]
    lsems = [ls0, ls1, ls2, ls3]
    esems = [es0, es1, es2, es3]
    ssems = [ss0, ss1, ss2, ss3]

    wid = lax.axis_index("s") * NUM_CORES + lax.axis_index("c")
    base = wid * SEQ_PER_WORKER

    def issue_load(s):
        j = s % 4
        row = base + s * CHUNK_ROWS
        return (pltpu.async_copy(
                    x_hbm.at[0, pl.ds(row, CHUNK_ROWS), :], xbufs[j],
                    lsems[j]),
                pltpu.async_copy(
                    emb_hbm.at[pl.ds(row, CHUNK_ROWS), :], ebufs[j],
                    esems[j]))

    def issue_store(s):
        j = s % 4
        row = base + s * CHUNK_ROWS
        return pltpu.async_copy(
            xbufs[j], out_hbm.at[0, pl.ds(row, CHUNK_ROWS), :], ssems[j])

    pending_load = [None] * 4
    pending_store = [None] * 4

    pending_load[0] = issue_load(0)
    pending_load[1] = issue_load(1)

    for s in range(NUM_CHUNKS):
        j = s % 4
        if s + 2 < NUM_CHUNKS:
            if pending_store[(s + 2) % 4] is not None:
                pending_store[(s + 2) % 4].wait()
                pending_store[(s + 2) % 4] = None
            pending_load[(s + 2) % 4] = issue_load(s + 2)
        for c in pending_load[j]:
            c.wait()
        pending_load[j] = None

        xv = xbufs[j]
        ev = ebufs[j]

        @plsc.parallel_loop(0, D_MODEL, LANES, unroll=UNROLL)
        def add_body(c, xv=xv, ev=ev):
            for r in range(CHUNK_ROWS):
                plsc.addupdate(xv.at[r, pl.ds(c, LANES)],
                               ev[r, pl.ds(c, LANES)])

        pending_store[j] = issue_store(s)

    for j in range(4):
        if pending_store[j] is not None:
            pending_store[j].wait()


def _tc_body(x_ref, emb_ref, o_ref):
    o_ref[...] = x_ref[...] + emb_ref[...]


@jax.jit
def _pos_emb_add(x, emb_table):
    mesh = plsc.VectorSubcoreMesh(core_axis_name="c", subcore_axis_name="s")
    sc_fn = pl.kernel(
        _sc_body,
        mesh=mesh,
        out_type=jax.ShapeDtypeStruct((1, SEQ_LEN, D_MODEL), jnp.float32),
        scratch_types=[pltpu.VMEM((CHUNK_ROWS, D_MODEL), jnp.float32)] * 8
        + [pltpu.SemaphoreType.DMA] * 12,
    )
    sc_out = sc_fn(x, emb_table)

    tc_out = pl.pallas_call(
        _tc_body,
        grid=(SEQ_LEN // TC_BS, BATCH - 1),
        in_specs=[
            pl.BlockSpec((1, TC_BS, D_MODEL), lambda i, b: (b + 1, i, 0)),
            pl.BlockSpec((TC_BS, D_MODEL), lambda i, b: (i, 0)),
        ],
        out_specs=pl.BlockSpec((1, TC_BS, D_MODEL), lambda i, b: (b, i, 0)),
        out_shape=jax.ShapeDtypeStruct((BATCH - 1, SEQ_LEN, D_MODEL),
                                       jnp.float32),
    )(x, emb_table)

    return lax.concatenate([sc_out, tc_out], 0)


def kernel(x, emb_table):
    return _pos_emb_add(x, emb_table[: x.shape[1]])


# speedup vs baseline: 2.3503x; 2.3503x over previous
import jax
import jax.numpy as jnp
from jax.experimental import pallas as pl

D_MODEL = 1024
SEQ_LEN = 4096
BATCH = 4
TC_BS = 512


def _tc_body(x_ref, emb_ref, o_ref):
    o_ref[...] = x_ref[...] + emb_ref[...]


@jax.jit
def _pos_emb_add(x, emb_table):
    return pl.pallas_call(
        _tc_body,
        grid=(SEQ_LEN // TC_BS,),
        in_specs=[
            pl.BlockSpec((BATCH, TC_BS, D_MODEL), lambda i: (0, i, 0)),
            pl.BlockSpec((TC_BS, D_MODEL), lambda i: (i, 0)),
        ],
        out_specs=pl.BlockSpec((BATCH, TC_BS, D_MODEL), lambda i: (0, i, 0)),
        out_shape=jax.ShapeDtypeStruct((BATCH, SEQ_LEN, D_MODEL), jnp.float32),
    )(x, emb_table)


def kernel(x, emb_table):
    return _pos_emb_add(x, emb_table[: x.shape[1]])


# speedup vs baseline: 2.3843x; 1.0145x over previous
import jax
import jax.numpy as jnp
from jax.experimental import pallas as pl

D_MODEL = 1024
SEQ_LEN = 4096
BATCH = 4
TC_BS = 2048


def _tc_body(x_ref, emb_ref, o_ref):
    o_ref[...] = x_ref[...] + emb_ref[...]


@jax.jit
def _pos_emb_add(x, emb_table):
    return pl.pallas_call(
        _tc_body,
        grid=(SEQ_LEN // TC_BS, BATCH),
        in_specs=[
            pl.BlockSpec((1, TC_BS, D_MODEL), lambda i, b: (b, i, 0)),
            pl.BlockSpec((TC_BS, D_MODEL), lambda i, b: (i, 0)),
        ],
        out_specs=pl.BlockSpec((1, TC_BS, D_MODEL), lambda i, b: (b, i, 0)),
        out_shape=jax.ShapeDtypeStruct((BATCH, SEQ_LEN, D_MODEL), jnp.float32),
    )(x, emb_table)


def kernel(x, emb_table):
    return _pos_emb_add(x, emb_table[: x.shape[1]])
